# baseline (speedup 1.0000x reference)
import functools

import jax
import jax.numpy as jnp
from jax import lax
from jax.experimental import pallas as pl
from jax.experimental.pallas import tpu as pltpu
from jax.experimental.pallas import tpu_sc as plsc

_NUM_E = 1000000
_H = 64
_BATCH = 16384
_NW = 32
_BPW = _BATCH // _NW
_CHUNK = 128
_NCHUNK = _BPW // _CHUNK

_BLK = 40960
_NBLK = (_NUM_E + _BLK - 1) // _BLK
_YPAD = _NBLK * _BLK


def _matvec_body(w_ref, b_ref, tbl_ref, y_ref):
    x = jnp.dot(w_ref[...], tbl_ref[...]) + b_ref[0, 0]
    y_ref[...] = (1.0 / (1.0 + jnp.exp(-x))).reshape(_BLK)


@jax.jit
def _gate_all_tc(tableT, w_row, b2):
    return pl.pallas_call(
        _matvec_body,
        grid=(_NBLK,),
        in_specs=[
            pl.BlockSpec((1, _H), lambda j: (0, 0)),
            pl.BlockSpec((1, 1), lambda j: (0, 0)),
            pl.BlockSpec((_H, _BLK), lambda j: (0, j)),
        ],
        out_specs=pl.BlockSpec((_BLK,), lambda j: (j,)),
        out_shape=jax.ShapeDtypeStruct((_YPAD,), jnp.float32),
    )(w_row, b2, tableT)


def _gather_body(y_hbm, idx_hbm, out_hbm, idx_v, g_v, sem):
    wid = lax.axis_index("s") * 2 + lax.axis_index("c")

    pltpu.sync_copy(idx_hbm.at[wid], idx_v)
    copies = [
        pltpu.async_copy(
            y_hbm.at[idx_v.at[j]],
            g_v.at[j],
            sem,
        )
        for j in range(_NCHUNK)
    ]
    for c in copies:
        c.wait()
    pltpu.sync_copy(g_v, out_hbm.at[wid])


@jax.jit
def _gather_sc(y1d, idx):
    mesh = plsc.VectorSubcoreMesh(core_axis_name="c", subcore_axis_name="s")
    f = functools.partial(
        pl.kernel,
        mesh=mesh,
        out_type=jax.ShapeDtypeStruct((_NW, _NCHUNK, _CHUNK), jnp.float32),
        scratch_types=[
            pltpu.VMEM((_NCHUNK, _CHUNK), jnp.int32),
            pltpu.VMEM((_NCHUNK, _CHUNK), jnp.float32),
            pltpu.SemaphoreType.DMA,
        ],
    )(_gather_body)
    return f(y1d, idx)


def kernel(X, Y, gate_theta, W, b):
    w_row = W.reshape(1, _H)
    b2 = b.reshape(1, 1)
    y = _gate_all_tc(gate_theta.T, w_row, b2)
    idx = X.reshape(_NW, _NCHUNK, _CHUNK)
    return _gather_sc(y, idx).reshape(_BATCH, 1)


# speedup vs baseline: 1.0090x; 1.0090x over previous
import functools

import jax
import jax.numpy as jnp
from jax import lax
from jax.experimental import pallas as pl
from jax.experimental.pallas import tpu as pltpu
from jax.experimental.pallas import tpu_sc as plsc

_NUM_E = 1000000
_H = 64
_BATCH = 16384
_NW = 32
_BPW = _BATCH // _NW
_CHUNK = 128
_NCHUNK = _BPW // _CHUNK

_BLK = 32768
_NBLK = (_NUM_E + _BLK - 1) // _BLK
_YPAD = _NBLK * _BLK


def _matvec_body(w_ref, b_ref, tbl_ref, y_ref):
    x = jnp.dot(w_ref[...], tbl_ref[...]) + b_ref[0, 0]
    y_ref[...] = (1.0 / (1.0 + jnp.exp(-x))).reshape(_BLK)


@jax.jit
def _gate_all_tc(tableT, w_row, b2):
    return pl.pallas_call(
        _matvec_body,
        grid=(_NBLK,),
        in_specs=[
            pl.BlockSpec((1, _H), lambda j: (0, 0)),
            pl.BlockSpec((1, 1), lambda j: (0, 0)),
            pl.BlockSpec((_H, _BLK), lambda j: (0, j)),
        ],
        out_specs=pl.BlockSpec((_BLK,), lambda j: (j,)),
        out_shape=jax.ShapeDtypeStruct((_YPAD,), jnp.float32),
    )(w_row, b2, tableT)


def _gather_body(y_hbm, idx_hbm, out_hbm, idx_v, g_v, sem):
    wid = lax.axis_index("s") * 2 + lax.axis_index("c")

    pltpu.sync_copy(idx_hbm.at[wid], idx_v)
    copies = [
        pltpu.async_copy(
            y_hbm.at[idx_v.at[j]],
            g_v.at[j],
            sem,
        )
        for j in range(_NCHUNK)
    ]
    for c in copies:
        c.wait()
    pltpu.sync_copy(g_v, out_hbm.at[wid])


@jax.jit
def _gather_sc(y1d, idx):
    mesh = plsc.VectorSubcoreMesh(core_axis_name="c", subcore_axis_name="s")
    f = functools.partial(
        pl.kernel,
        mesh=mesh,
        out_type=jax.ShapeDtypeStruct((_NW, _NCHUNK, _CHUNK), jnp.float32),
        scratch_types=[
            pltpu.VMEM((_NCHUNK, _CHUNK), jnp.int32),
            pltpu.VMEM((_NCHUNK, _CHUNK), jnp.float32),
            pltpu.SemaphoreType.DMA,
        ],
    )(_gather_body)
    return f(y1d, idx)


def kernel(X, Y, gate_theta, W, b):
    w_row = W.reshape(1, _H)
    b2 = b.reshape(1, 1)
    y = _gate_all_tc(gate_theta.T, w_row, b2)
    idx = X.reshape(_NW, _NCHUNK, _CHUNK)
    return _gather_sc(y, idx).reshape(_BATCH, 1)
